# baseline (speedup 1.0000x reference)
import functools

import jax
import jax.numpy as jnp
from jax import lax
from jax.experimental import pallas as pl
from jax.experimental.pallas import tpu as pltpu
from jax.experimental.pallas import tpu_sc as plsc

_B, _S = 4096, 200
_D = 32
_NC, _NS = 2, 16
_NW = _NC * _NS
_CHUNK = 1024
_CPS = _B // _CHUNK
_NUNIT = _S * _CPS
_PER_W = _NUNIT // _NW

_mesh = plsc.VectorSubcoreMesh(core_axis_name="c", subcore_axis_name="s")


@functools.partial(
    pl.kernel,
    mesh=_mesh,
    out_type=jax.ShapeDtypeStruct((_S, _B, _D), jnp.float32),
    scratch_types=[
        pltpu.VMEM((2, _CHUNK), jnp.int32),
        pltpu.VMEM((_CHUNK, _D), jnp.float32),
        pltpu.VMEM((_CHUNK, _D), jnp.float32),
        pltpu.SemaphoreType.DMA,
        pltpu.SemaphoreType.DMA,
        pltpu.SemaphoreType.DMA,
        pltpu.SemaphoreType.DMA,
        pltpu.SemaphoreType.DMA,
        pltpu.SemaphoreType.DMA,
    ],
    compiler_params=pltpu.CompilerParams(use_tc_tiling_on_sc=False),
)
def _gather_kernel(idx_hbm, table_hbm, out_hbm, idx_v, rows0, rows1,
                   isem0, isem1, gsem0, gsem1, wsem0, wsem1):
    wid = lax.axis_index("s") * _NC + lax.axis_index("c")
    u0 = wid * _PER_W

    rows = (rows0, rows1)
    isem = (isem0, isem1)
    gsem = (gsem0, gsem1)
    wsem = (wsem0, wsem1)

    def idx_src(u):
        s, c = u // _CPS, u % _CPS
        return idx_hbm.at[s, pl.ds(c * _CHUNK, _CHUNK)]

    def start_idx(j, b):
        pltpu.async_copy(idx_src(u0 + j), idx_v.at[b], isem[b])

    def gather(j, b):
        pltpu.make_async_copy(idx_src(u0), idx_v.at[b], isem[b]).wait()
        pltpu.async_copy(table_hbm.at[idx_v.at[b]], rows[b], gsem[b])

    def start_write(j, b):
        u = u0 + j
        s, c = u // _CPS, u % _CPS
        pltpu.make_async_copy(table_hbm.at[idx_v.at[b]], rows[b],
                              gsem[b]).wait()
        pltpu.async_copy(rows[b], out_hbm.at[s, pl.ds(c * _CHUNK, _CHUNK)],
                         wsem[b])

    def wait_write(b):
        pltpu.make_async_copy(rows[b], out_hbm.at[0, pl.ds(0, _CHUNK)],
                              wsem[b]).wait()

    start_idx(0, 0)
    gather(0, 0)
    start_idx(1, 1)

    def body(j, carry):
        b = lax.rem(j, 2)

        @pl.when(b == 0)
        def _():
            start_write(j, 0)

            @pl.when(j + 1 < _PER_W)
            def _():
                gather(j + 1, 1)

            @pl.when(j + 2 < _PER_W)
            def _():
                wait_write(0)
                start_idx(j + 2, 0)

        @pl.when(b == 1)
        def _():
            start_write(j, 1)

            @pl.when(j + 1 < _PER_W)
            def _():
                gather(j + 1, 0)

            @pl.when(j + 2 < _PER_W)
            def _():
                wait_write(1)
                start_idx(j + 2, 1)

        return carry

    lax.fori_loop(0, _PER_W, body, 0)
    wait_write((_PER_W - 1) % 2)
    wait_write(_PER_W % 2)


_E = 1_000_000
_TCOLS = 4096
_TGRID = -(-_E // _TCOLS)


def _table_format_body(in_ref, out_ref):
    x = in_ref[...]
    x = jnp.reshape(x, (_D, _TCOLS // 4, 4))
    x = jnp.transpose(x, (1, 2, 0))
    out_ref[...] = jnp.reshape(x, (_TCOLS // 4, 128))


_table_format = pl.pallas_call(
    _table_format_body,
    grid=(_TGRID,),
    in_specs=[pl.BlockSpec((_D, _TCOLS), lambda i: (0, i))],
    out_specs=pl.BlockSpec((_TCOLS // 4, 128), lambda i: (i, 0)),
    out_shape=jax.ShapeDtypeStruct((_E * _D // 128, 128), jnp.float32),
)


def kernel(entity_indices, entity_embeddings):
    idx_t = jnp.transpose(entity_indices)
    table_lin = _table_format(jnp.transpose(entity_embeddings))
    table2 = jnp.reshape(table_lin, (_E, _D))
    out = _gather_kernel(idx_t, table2)
    return jnp.transpose(out, (1, 0, 2))


# speedup vs baseline: 3.0653x; 3.0653x over previous
import functools

import jax
import jax.numpy as jnp
from jax import lax
from jax.experimental import pallas as pl
from jax.experimental.pallas import tpu as pltpu
from jax.experimental.pallas import tpu_sc as plsc

_B, _S = 4096, 200
_D = 32
_NC, _NS = 2, 16
_NW = _NC * _NS
_CHUNK = 1024
_CPS = _B // _CHUNK
_NUNIT = _S * _CPS
_PER_W = _NUNIT // _NW

_mesh = plsc.VectorSubcoreMesh(core_axis_name="c", subcore_axis_name="s")


@functools.partial(
    pl.kernel,
    mesh=_mesh,
    out_type=jax.ShapeDtypeStruct((_S, _B, _D), jnp.float32),
    scratch_types=[
        pltpu.VMEM((2, _CHUNK), jnp.int32),
        pltpu.VMEM((_CHUNK, _D), jnp.float32),
        pltpu.VMEM((_CHUNK, _D), jnp.float32),
        pltpu.SemaphoreType.DMA,
        pltpu.SemaphoreType.DMA,
        pltpu.SemaphoreType.DMA,
        pltpu.SemaphoreType.DMA,
        pltpu.SemaphoreType.DMA,
        pltpu.SemaphoreType.DMA,
    ],
    compiler_params=pltpu.CompilerParams(use_tc_tiling_on_sc=False),
)
def _gather_kernel(idx_hbm, table_hbm, out_hbm, idx_v, rows0, rows1,
                   isem0, isem1, gsem0, gsem1, wsem0, wsem1):
    wid = lax.axis_index("s") * _NC + lax.axis_index("c")
    u0 = wid * _PER_W

    rows = (rows0, rows1)
    isem = (isem0, isem1)
    gsem = (gsem0, gsem1)
    wsem = (wsem0, wsem1)

    def idx_src(u):
        s, c = u // _CPS, u % _CPS
        return idx_hbm.at[s, pl.ds(c * _CHUNK, _CHUNK)]

    def start_idx(j, b):
        pltpu.async_copy(idx_src(u0 + j), idx_v.at[b], isem[b])

    def gather(j, b):
        pltpu.make_async_copy(idx_src(u0), idx_v.at[b], isem[b]).wait()
        pltpu.async_copy(table_hbm.at[idx_v.at[b]], rows[b], gsem[b])

    def start_write(j, b):
        u = u0 + j
        s, c = u // _CPS, u % _CPS
        pltpu.make_async_copy(table_hbm.at[idx_v.at[b]], rows[b],
                              gsem[b]).wait()
        pltpu.async_copy(rows[b], out_hbm.at[s, pl.ds(c * _CHUNK, _CHUNK)],
                         wsem[b])

    def wait_write(b):
        pltpu.make_async_copy(rows[b], out_hbm.at[0, pl.ds(0, _CHUNK)],
                              wsem[b]).wait()

    start_idx(0, 0)
    gather(0, 0)
    start_idx(1, 1)

    def body(j, carry):
        b = lax.rem(j, 2)

        @pl.when(b == 0)
        def _():
            start_write(j, 0)

            @pl.when(j + 1 < _PER_W)
            def _():
                gather(j + 1, 1)

            @pl.when(j + 2 < _PER_W)
            def _():
                wait_write(0)
                start_idx(j + 2, 0)

        @pl.when(b == 1)
        def _():
            start_write(j, 1)

            @pl.when(j + 1 < _PER_W)
            def _():
                gather(j + 1, 0)

            @pl.when(j + 2 < _PER_W)
            def _():
                wait_write(1)
                start_idx(j + 2, 1)

        return carry

    lax.fori_loop(0, _PER_W, body, 0)
    wait_write((_PER_W - 1) % 2)
    wait_write(_PER_W % 2)


def kernel(entity_indices, entity_embeddings):
    idx_t = jnp.transpose(entity_indices)
    out = _gather_kernel(idx_t, entity_embeddings)
    return jnp.transpose(out, (1, 0, 2))
